# Initial kernel scaffold; baseline (speedup 1.0000x reference)
import jax
import jax.numpy as jnp
from jax.experimental import pallas as pl


def kernel(inputs, embed):
    raise NotImplementedError("write your pallas kernel here")


# speedup vs baseline: 1.1491x; 1.1491x over previous
import functools

import jax
import jax.numpy as jnp
from jax import lax
from jax.experimental import pallas as pl
from jax.experimental.pallas import tpu as pltpu
from jax.experimental.pallas import tpu_sc as plsc

DIM = 256
K = 8192
COMMIT = 0.25

N_ROWS = 16 * 576

RT = 1152
KT = 1024
NR = N_ROWS // RT
NK = K // KT

FT = 1152
NF = N_ROWS // FT
HIST_CHUNK = 512


def _stats_body(x_ref, mu_ref, sd_ref):
    x = x_ref[...]
    mu = jnp.mean(x, axis=0, keepdims=True)
    d = x - mu
    var = jnp.sum(d * d, axis=0, keepdims=True) / (N_ROWS - 1)
    mu_ref[...] = mu
    sd_ref[...] = jnp.sqrt(var)


def _stats(x):
    return pl.pallas_call(
        _stats_body,
        out_shape=(
            jax.ShapeDtypeStruct((1, DIM), jnp.float32),
            jax.ShapeDtypeStruct((1, DIM), jnp.float32),
        ),
    )(x)


def _prep_body(e_ref, mu_ref, sd_ref, es_ref, e2_ref):
    es = e_ref[...] * sd_ref[...] * 0.5 + mu_ref[...]
    es_ref[...] = es
    e2_ref[...] = jnp.sum(es * es, axis=1, keepdims=True)


def _prep(embed, mu, sd):
    return pl.pallas_call(
        _prep_body,
        out_shape=(
            jax.ShapeDtypeStruct((K, DIM), jnp.float32),
            jax.ShapeDtypeStruct((K, 1), jnp.float32),
        ),
    )(embed, mu, sd)


def _argmin_body(x_ref, es_ref, e2_ref, idx_ref, minv_ref):
    k = pl.program_id(1)
    m = lax.dot_general(x_ref[...], es_ref[...], (((1,), (1,)), ((), ())),
                        preferred_element_type=jnp.float32)
    score = e2_ref[...] - 2.0 * m
    local_min = jnp.min(score, axis=1, keepdims=True)
    lanes = lax.broadcasted_iota(jnp.int32, score.shape, 1)
    local_arg = jnp.min(
        jnp.where(score == local_min, lanes, jnp.int32(2**30)),
        axis=1, keepdims=True) + k * KT

    @pl.when(k == 0)
    def _():
        minv_ref[...] = local_min
        idx_ref[...] = local_arg

    @pl.when(k > 0)
    def _():
        better = local_min < minv_ref[...]
        minv_ref[...] = jnp.where(better, local_min, minv_ref[...])
        idx_ref[...] = jnp.where(better, local_arg, idx_ref[...])


def _argmin(x, es, e2row):
    return pl.pallas_call(
        _argmin_body,
        grid=(NR, NK),
        in_specs=[
            pl.BlockSpec((RT, DIM), lambda r, k: (r, 0)),
            pl.BlockSpec((KT, DIM), lambda r, k: (k, 0)),
            pl.BlockSpec((1, KT), lambda r, k: (0, k)),
        ],
        out_specs=pl.BlockSpec((RT, 1), lambda r, k: (r, 0)),
        out_shape=jax.ShapeDtypeStruct((N_ROWS, 1), jnp.int32),
        scratch_shapes=[pltpu.VMEM((RT, 1), jnp.float32)],
    )(x, es, e2row)


def _gather(table, idx):
    info = plsc.get_sparse_core_info()
    nc, ns = info.num_cores, info.num_subcores
    nw = nc * ns
    b_per_w = N_ROWS // nw

    @functools.partial(
        pl.kernel,
        mesh=plsc.VectorSubcoreMesh(core_axis_name="c", subcore_axis_name="s"),
        out_type=jax.ShapeDtypeStruct((N_ROWS, DIM), jnp.float32),
        scratch_types=[
            pltpu.VMEM((b_per_w,), jnp.int32),
            pltpu.VMEM((b_per_w, DIM), jnp.float32),
            pltpu.SemaphoreType.DMA,
        ],
    )
    def gather_k(table_hbm, idx_hbm, out_hbm, idx_v, rows_v, sem):
        wid = lax.axis_index("s") * nc + lax.axis_index("c")
        base = wid * b_per_w
        pltpu.sync_copy(idx_hbm.at[pl.ds(base, b_per_w)], idx_v)
        pltpu.async_copy(table_hbm.at[idx_v], rows_v, sem).wait()
        pltpu.sync_copy(rows_v, out_hbm.at[pl.ds(base, b_per_w)])

    return gather_k(table, idx)


def _finalize_body(x_ref, g_ref, idx_ref, q_ref, loss_ref, perp_ref, cnt_ref):
    r = pl.program_id(0)
    x = x_ref[...]
    q = g_ref[...]
    q_ref[...] = x + (q - x)
    d = x - q
    part = jnp.sum(d * d)

    @pl.when(r == 0)
    def _():
        loss_ref[...] = jnp.full((1, 1), part, jnp.float32)
        cnt_ref[...] = jnp.zeros_like(cnt_ref)
        perp_ref[...] = jnp.zeros((1, 1), jnp.float32)

    @pl.when(r > 0)
    def _():
        loss_ref[...] = loss_ref[...] + part

    idx = idx_ref[...]
    for c in range(K // HIST_CHUNK):
        bins = lax.broadcasted_iota(jnp.int32, (FT, HIST_CHUNK), 1) \
            + c * HIST_CHUNK
        onehot = (idx == bins).astype(jnp.float32)
        cnt_ref[:, c * HIST_CHUNK:(c + 1) * HIST_CHUNK] += jnp.sum(
            onehot, axis=0, keepdims=True)

    @pl.when(r == NF - 1)
    def _():
        loss_ref[...] = loss_ref[...] * (COMMIT / (N_ROWS * DIM))
        p = cnt_ref[...] / N_ROWS
        ent = -jnp.sum(p * jnp.log(p + 1e-10))
        perp_ref[...] = jnp.full((1, 1), jnp.exp(ent), jnp.float32)


def _finalize(x, g, idx):
    return pl.pallas_call(
        _finalize_body,
        grid=(NF,),
        in_specs=[
            pl.BlockSpec((FT, DIM), lambda r: (r, 0)),
            pl.BlockSpec((FT, DIM), lambda r: (r, 0)),
            pl.BlockSpec((FT, 1), lambda r: (r, 0)),
        ],
        out_specs=(
            pl.BlockSpec((FT, DIM), lambda r: (r, 0)),
            pl.BlockSpec((1, 1), lambda r: (0, 0)),
            pl.BlockSpec((1, 1), lambda r: (0, 0)),
        ),
        out_shape=(
            jax.ShapeDtypeStruct((N_ROWS, DIM), jnp.float32),
            jax.ShapeDtypeStruct((1, 1), jnp.float32),
            jax.ShapeDtypeStruct((1, 1), jnp.float32),
        ),
        scratch_shapes=[pltpu.VMEM((1, K), jnp.float32)],
    )(x, g, idx)


def kernel(inputs, embed):
    B, D, T = inputs.shape
    x = jnp.transpose(inputs, (0, 2, 1)).reshape(N_ROWS, D)
    mu, sd = _stats(x)
    es, e2col = _prep(embed, mu, sd)
    e2row = e2col.reshape(1, K)
    idx2d = _argmin(x, es, e2row)
    idx = idx2d.reshape(N_ROWS)
    g = _gather(es, idx)
    q_flat, loss, perp = _finalize(x, g, idx2d)
    quantized = jnp.transpose(q_flat.reshape(B, T, D), (0, 2, 1))
    return (quantized, loss.reshape(()), perp.reshape(()))


# speedup vs baseline: 1.3713x; 1.1934x over previous
import functools

import jax
import jax.numpy as jnp
from jax import lax
from jax.experimental import pallas as pl
from jax.experimental.pallas import tpu as pltpu
from jax.experimental.pallas import tpu_sc as plsc

DIM = 256
K = 8192
COMMIT = 0.25

N_ROWS = 16 * 576

RT = 1152
KT = 1024
NR = N_ROWS // RT
NK = K // KT

HI = 128
LO = 64


def _prep_body(x_ref, e_ref, es_ref, e2_ref, x2_ref):
    x = x_ref[...]
    mu = jnp.mean(x, axis=0, keepdims=True)
    d = x - mu
    var = jnp.sum(d * d, axis=0, keepdims=True) / (N_ROWS - 1)
    sd = jnp.sqrt(var)
    es = e_ref[...] * sd * 0.5 + mu
    es_ref[...] = es
    e2_ref[...] = jnp.sum(es * es, axis=1, keepdims=True)
    x2_ref[...] = jnp.sum(x * x, axis=1, keepdims=True)


def _prep(x, embed):
    return pl.pallas_call(
        _prep_body,
        out_shape=(
            jax.ShapeDtypeStruct((K, DIM), jnp.float32),
            jax.ShapeDtypeStruct((K, 1), jnp.float32),
            jax.ShapeDtypeStruct((N_ROWS, 1), jnp.float32),
        ),
    )(x, embed)


def _argmin_body(x_ref, es_ref, e2_ref, x2_ref, idx_ref, loss_ref, minv_ref):
    r = pl.program_id(0)
    k = pl.program_id(1)
    m = lax.dot_general(x_ref[...], es_ref[...], (((1,), (1,)), ((), ())),
                        preferred_element_type=jnp.float32)
    score = e2_ref[...] - 2.0 * m
    local_min = jnp.min(score, axis=1, keepdims=True)
    lanes = lax.broadcasted_iota(jnp.int32, score.shape, 1)
    local_arg = jnp.min(
        jnp.where(score == local_min, lanes, jnp.int32(2**30)),
        axis=1, keepdims=True) + k * KT

    @pl.when(k == 0)
    def _():
        minv_ref[...] = local_min
        idx_ref[...] = local_arg

    @pl.when(k > 0)
    def _():
        better = local_min < minv_ref[...]
        minv_ref[...] = jnp.where(better, local_min, minv_ref[...])
        idx_ref[...] = jnp.where(better, local_arg, idx_ref[...])

    @pl.when(k == NK - 1)
    def _():
        part = jnp.sum(minv_ref[...] + x2_ref[...])

        @pl.when(r == 0)
        def _():
            loss_ref[...] = jnp.full((1, 1), part, jnp.float32)

        @pl.when(r > 0)
        def _():
            loss_ref[...] = loss_ref[...] + part

        @pl.when(r == NR - 1)
        def _():
            loss_ref[...] = loss_ref[...] * (COMMIT / (N_ROWS * DIM))


def _argmin(x, es, e2row, x2):
    return pl.pallas_call(
        _argmin_body,
        grid=(NR, NK),
        in_specs=[
            pl.BlockSpec((RT, DIM), lambda r, k: (r, 0)),
            pl.BlockSpec((KT, DIM), lambda r, k: (k, 0)),
            pl.BlockSpec((1, KT), lambda r, k: (0, k)),
            pl.BlockSpec((RT, 1), lambda r, k: (r, 0)),
        ],
        out_specs=(
            pl.BlockSpec((RT, 1), lambda r, k: (r, 0)),
            pl.BlockSpec((1, 1), lambda r, k: (0, 0)),
        ),
        out_shape=(
            jax.ShapeDtypeStruct((N_ROWS, 1), jnp.int32),
            jax.ShapeDtypeStruct((1, 1), jnp.float32),
        ),
        scratch_shapes=[pltpu.VMEM((RT, 1), jnp.float32)],
    )(x, es, e2row, x2)


def _gather(table, idx):
    info = plsc.get_sparse_core_info()
    nc, ns = info.num_cores, info.num_subcores
    nw = nc * ns
    b_per_w = N_ROWS // nw

    @functools.partial(
        pl.kernel,
        mesh=plsc.VectorSubcoreMesh(core_axis_name="c", subcore_axis_name="s"),
        out_type=jax.ShapeDtypeStruct((N_ROWS, DIM), jnp.float32),
        scratch_types=[
            pltpu.VMEM((b_per_w,), jnp.int32),
            pltpu.VMEM((b_per_w, DIM), jnp.float32),
            pltpu.SemaphoreType.DMA,
        ],
    )
    def gather_k(table_hbm, idx_hbm, out_hbm, idx_v, rows_v, sem):
        wid = lax.axis_index("s") * nc + lax.axis_index("c")
        base = wid * b_per_w
        pltpu.sync_copy(idx_hbm.at[pl.ds(base, b_per_w)], idx_v)
        pltpu.async_copy(table_hbm.at[idx_v], rows_v, sem).wait()
        pltpu.sync_copy(rows_v, out_hbm.at[pl.ds(base, b_per_w)])

    return gather_k(table, idx)


def _hist_body(idx_ref, perp_ref):
    idx = idx_ref[...]
    hi = lax.shift_right_logical(idx, 6)
    lo = lax.bitwise_and(idx, jnp.int32(LO - 1))
    hb = lax.broadcasted_iota(jnp.int32, (N_ROWS, HI), 1)
    lb = lax.broadcasted_iota(jnp.int32, (N_ROWS, LO), 1)
    H = (hi == hb).astype(jnp.float32)
    L = (lo == lb).astype(jnp.float32)
    counts = lax.dot_general(H, L, (((0,), (0,)), ((), ())),
                             preferred_element_type=jnp.float32)
    p = counts / N_ROWS
    ent = -jnp.sum(p * jnp.log(p + 1e-10))
    perp_ref[...] = jnp.full((1, 1), jnp.exp(ent), jnp.float32)


def _hist(idx2d):
    return pl.pallas_call(
        _hist_body,
        out_shape=jax.ShapeDtypeStruct((1, 1), jnp.float32),
    )(idx2d)


def kernel(inputs, embed):
    B, D, T = inputs.shape
    x = jnp.transpose(inputs, (0, 2, 1)).reshape(N_ROWS, D)
    es, e2col, x2 = _prep(x, embed)
    e2row = e2col.reshape(1, K)
    idx2d, loss = _argmin(x, es, e2row, x2)
    idx = idx2d.reshape(N_ROWS)
    q_flat = _gather(es, idx)
    perp = _hist(idx2d)
    quantized = jnp.transpose(q_flat.reshape(B, T, D), (0, 2, 1))
    return (quantized, loss.reshape(()), perp.reshape(()))
